# baseline (speedup 1.0000x reference)
import functools

import jax
import jax.numpy as jnp
from jax import lax
from jax.experimental import pallas as pl
from jax.experimental.pallas import tpu as pltpu
from jax.experimental.pallas import tpu_sc as plsc

_B, _N, _M, _HA, _HB = 8, 1024, 16, 128, 16
_C = 2 * _HA
_ROWS = _B * _N * _M
_NODES = _B * _N
_TN = 1024
_TR = _TN * _M
_INV_NH = 1.0 / _ROWS
_INV_NO = 1.0 / _NODES
_EPS = 1e-5


def _proj_body(x_ref, wf_ref, wc_ref, o_ref):
    yf = jnp.dot(x_ref[...], wf_ref[...], preferred_element_type=jnp.float32)
    yc = jnp.dot(x_ref[...], wc_ref[...], preferred_element_type=jnp.float32)
    uf = lax.bitcast_convert_type(yf.astype(jnp.bfloat16), jnp.uint16).astype(jnp.int32)
    uc = lax.bitcast_convert_type(yc.astype(jnp.bfloat16), jnp.uint16).astype(jnp.int32)
    o_ref[...] = uf | (uc << 16)


def _tc_proj(x, WnTf, WnTc):
    T = 1024
    return pl.pallas_call(
        _proj_body,
        grid=(_NODES // T,),
        in_specs=[pl.BlockSpec((T, _HA), lambda i: (i, 0)),
                  pl.BlockSpec((_HA, _HA), lambda i: (0, 0)),
                  pl.BlockSpec((_HA, _HA), lambda i: (0, 0))],
        out_specs=pl.BlockSpec((T, _HA), lambda i: (i, 0)),
        out_shape=jax.ShapeDtypeStruct((_NODES, _HA), jnp.int32),
    )(x, WnTf, WnTc)


def _z_tile(g_ref, x_ref, nbr_ref, wst_ref, wet_ref, bfc_ref):
    S = jnp.dot(x_ref[...], wst_ref[...], preferred_element_type=jnp.float32) + bfc_ref[...]
    E = jnp.dot(nbr_ref[...].reshape(_TR, _HB), wet_ref[...],
                preferred_element_type=jnp.float32)
    g32 = g_ref[...]
    gf = lax.bitcast_convert_type(g32 << 16, jnp.float32).reshape(_TN, _M, _HA)
    gc = lax.bitcast_convert_type(g32 & jnp.int32(-65536), jnp.float32).reshape(_TN, _M, _HA)
    zf = gf + E[:, :_HA].reshape(_TN, _M, _HA) + S[:, :_HA][:, None, :]
    zc = gc + E[:, _HA:].reshape(_TN, _M, _HA) + S[:, _HA:][:, None, :]
    return zf.reshape(_TR, _HA), zc.reshape(_TR, _HA)


def _stats_body(g_ref, x_ref, nbr_ref, wst_ref, wet_ref, bfc_ref, acc_ref):
    i = pl.program_id(0)
    zf, zc = _z_tile(g_ref, x_ref, nbr_ref, wst_ref, wet_ref, bfc_ref)
    part = jnp.stack([jnp.concatenate([zf.sum(0), zc.sum(0)]),
                      jnp.concatenate([(zf * zf).sum(0), (zc * zc).sum(0)])])

    @pl.when(i == 0)
    def _():
        acc_ref[...] = jnp.zeros_like(acc_ref)

    acc_ref[...] += part


def _tc_stats(G, x, nbr4, WsT, WeT, bfc):
    return pl.pallas_call(
        _stats_body,
        grid=(_ROWS // _TR,),
        in_specs=[pl.BlockSpec((_TR, _HA), lambda i: (i, 0)),
                  pl.BlockSpec((_TN, _HA), lambda i: (i, 0)),
                  pl.BlockSpec((1, _TN, _M, _HB),
                               lambda i: (i // (_N // _TN), i % (_N // _TN), 0, 0)),
                  pl.BlockSpec((_HA, _C), lambda i: (0, 0)),
                  pl.BlockSpec((_HB, _C), lambda i: (0, 0)),
                  pl.BlockSpec((1, _C), lambda i: (0, 0))],
        out_specs=pl.BlockSpec((2, _C), lambda i: (0, 0)),
        out_shape=jax.ShapeDtypeStruct((2, _C), jnp.float32),
    )(G, x, nbr4, WsT, WeT, bfc)


def _apply_body(g_ref, x_ref, nbr_ref, wst_ref, wet_ref, bfc_ref, st_ref,
                gam_ref, bet_ref, s_ref, acc_ref):
    i = pl.program_id(0)
    zf, zc = _z_tile(g_ref, x_ref, nbr_ref, wst_ref, wet_ref, bfc_ref)
    mean = st_ref[0, :] * _INV_NH
    var = st_ref[1, :] * _INV_NH - mean * mean
    scale = gam_ref[0, :] * lax.rsqrt(var + _EPS)
    shift = bet_ref[0, :] - mean * scale
    f = jax.nn.sigmoid(zf * scale[:_HA] + shift[:_HA])
    c = jnp.maximum(zc * scale[_HA:] + shift[_HA:], 0.0)
    fc = f * c
    s = fc.reshape(_TN, _M, _HA).sum(axis=1)
    s_ref[...] = s
    part = jnp.stack([s.sum(0), (s * s).sum(0)])

    @pl.when(i == 0)
    def _():
        acc_ref[...] = jnp.zeros_like(acc_ref)

    acc_ref[...] += part


def _tc_apply(G, x, nbr4, WsT, WeT, bfc, stats_h, gam, bet):
    return pl.pallas_call(
        _apply_body,
        grid=(_ROWS // _TR,),
        in_specs=[pl.BlockSpec((_TR, _HA), lambda i: (i, 0)),
                  pl.BlockSpec((_TN, _HA), lambda i: (i, 0)),
                  pl.BlockSpec((1, _TN, _M, _HB),
                               lambda i: (i // (_N // _TN), i % (_N // _TN), 0, 0)),
                  pl.BlockSpec((_HA, _C), lambda i: (0, 0)),
                  pl.BlockSpec((_HB, _C), lambda i: (0, 0)),
                  pl.BlockSpec((1, _C), lambda i: (0, 0)),
                  pl.BlockSpec((2, _C), lambda i: (0, 0)),
                  pl.BlockSpec((1, _C), lambda i: (0, 0)),
                  pl.BlockSpec((1, _C), lambda i: (0, 0))],
        out_specs=[pl.BlockSpec((_TN, _HA), lambda i: (i, 0)),
                   pl.BlockSpec((2, _HA), lambda i: (0, 0))],
        out_shape=[jax.ShapeDtypeStruct((_NODES, _HA), jnp.float32),
                   jax.ShapeDtypeStruct((2, _HA), jnp.float32)],
    )(G, x, nbr4, WsT, WeT, bfc, stats_h, gam, bet)


def _final_body(x_ref, s_ref, st_ref, gam_ref, bet_ref, o_ref):
    mean = st_ref[0, :] * _INV_NO
    var = st_ref[1, :] * _INV_NO - mean * mean
    scale = gam_ref[0, :] * lax.rsqrt(var + _EPS)
    shift = bet_ref[0, :] - mean * scale
    o_ref[...] = jnp.maximum(x_ref[...] + s_ref[...] * scale + shift, 0.0)


def _tc_final(x, s, stats_o, gam, bet):
    T = 1024
    return pl.pallas_call(
        _final_body,
        grid=(_NODES // T,),
        in_specs=[pl.BlockSpec((T, _HA), lambda i: (i, 0)),
                  pl.BlockSpec((T, _HA), lambda i: (i, 0)),
                  pl.BlockSpec((2, _HA), lambda i: (0, 0)),
                  pl.BlockSpec((1, _HA), lambda i: (0, 0)),
                  pl.BlockSpec((1, _HA), lambda i: (0, 0))],
        out_specs=pl.BlockSpec((T, _HA), lambda i: (i, 0)),
        out_shape=jax.ShapeDtypeStruct((_NODES, _HA), jnp.float32),
    )(x, s, stats_o, gam, bet)


def _sc_gather(table, gidx2):
    info = plsc.get_sparse_core_info()
    nw = info.num_cores * info.num_subcores
    per_w = _ROWS // nw
    ch = 128
    nch = per_w // ch
    gpc = 2
    nsc = nch // gpc
    mesh = plsc.VectorSubcoreMesh(core_axis_name="c", subcore_axis_name="s")

    @functools.partial(
        pl.kernel, mesh=mesh,
        out_type=jax.ShapeDtypeStruct((_ROWS, _HA), jnp.int32),
        scratch_types=[
            pltpu.VMEM((nch, ch), jnp.int32),
            pltpu.VMEM((2, gpc * ch, _HA), jnp.int32),
            pltpu.SemaphoreType.DMA,
            pltpu.SemaphoreType.DMA,
        ],
    )
    def k(table_hbm, idx_hbm, out_hbm, idx_v, rows_v, sem0, sem1):
        wid = lax.axis_index("s") * info.num_cores + lax.axis_index("c")
        base = wid * per_w
        pltpu.sync_copy(idx_hbm.at[pl.ds(wid * nch, nch)], idx_v)
        sems = (sem0, sem1)

        def start(i, j):
            for g in range(gpc):
                pltpu.async_copy(table_hbm.at[idx_v.at[i * gpc + g]],
                                 rows_v.at[j, pl.ds(g * ch, ch)], sems[j])

        def drain(i, j):
            for g in range(gpc):
                pltpu.make_async_copy(table_hbm.at[idx_v.at[i * gpc + g]],
                                      rows_v.at[j, pl.ds(g * ch, ch)],
                                      sems[j]).wait()

        start(0, 0)

        def body(i, carry):
            for j in range(2):
                @pl.when(i % 2 == j)
                def _():
                    @pl.when(i + 1 < nsc)
                    def _():
                        start(i + 1, 1 - j)

                    drain(i, j)
                    pltpu.sync_copy(
                        rows_v.at[j],
                        out_hbm.at[pl.ds(base + i * gpc * ch, gpc * ch)])
            return carry

        lax.fori_loop(0, nsc, body, 0)

    return k(table, gidx2)


def kernel(atom_emb, nbr_emb, nbr_adj_list, W_fc, b_fc, gamma_h, beta_h,
           gamma_o, beta_o):
    B, N, HA = atom_emb.shape
    M = nbr_adj_list.shape[2]
    C = 2 * HA
    x = atom_emb.reshape(B * N, HA)
    WsT = W_fc[:, :HA].T
    WnT = W_fc[:, HA:2 * HA].T
    WeT = W_fc[:, 2 * HA:].T

    P32 = _tc_proj(x, WnT[:, :HA], WnT[:, HA:])
    gidx = (nbr_adj_list.astype(jnp.int32)
            + (jnp.arange(B, dtype=jnp.int32) * N)[:, None, None]).reshape(-1, 128)
    G = _sc_gather(P32, gidx)

    nbr4 = nbr_emb
    bfc = b_fc.reshape(1, C)
    stats_h = _tc_stats(G, x, nbr4, WsT, WeT, bfc)
    s, stats_o = _tc_apply(G, x, nbr4, WsT, WeT, bfc, stats_h,
                           gamma_h.reshape(1, C), beta_h.reshape(1, C))
    out = _tc_final(x, s, stats_o, gamma_o.reshape(1, HA), beta_o.reshape(1, HA))
    return out.reshape(B, N, HA)


# speedup vs baseline: 1.0116x; 1.0116x over previous
import functools

import jax
import jax.numpy as jnp
from jax import lax
from jax.experimental import pallas as pl
from jax.experimental.pallas import tpu as pltpu
from jax.experimental.pallas import tpu_sc as plsc

_B, _N, _M, _HA, _HB = 8, 1024, 16, 128, 16
_C = 2 * _HA
_ROWS = _B * _N * _M
_NODES = _B * _N
_TN = 512
_TR = _TN * _M
_INV_NH = 1.0 / _ROWS
_INV_NO = 1.0 / _NODES
_EPS = 1e-5


def _proj_body(x_ref, wf_ref, wc_ref, o_ref):
    yf = jnp.dot(x_ref[...], wf_ref[...], preferred_element_type=jnp.float32)
    yc = jnp.dot(x_ref[...], wc_ref[...], preferred_element_type=jnp.float32)
    uf = lax.bitcast_convert_type(yf.astype(jnp.bfloat16), jnp.uint16).astype(jnp.int32)
    uc = lax.bitcast_convert_type(yc.astype(jnp.bfloat16), jnp.uint16).astype(jnp.int32)
    o_ref[...] = uf | (uc << 16)


def _tc_proj(x, WnTf, WnTc):
    T = 1024
    return pl.pallas_call(
        _proj_body,
        grid=(_NODES // T,),
        in_specs=[pl.BlockSpec((T, _HA), lambda i: (i, 0)),
                  pl.BlockSpec((_HA, _HA), lambda i: (0, 0)),
                  pl.BlockSpec((_HA, _HA), lambda i: (0, 0))],
        out_specs=pl.BlockSpec((T, _HA), lambda i: (i, 0)),
        out_shape=jax.ShapeDtypeStruct((_NODES, _HA), jnp.int32),
    )(x, WnTf, WnTc)


def _z_tile(g_ref, x_ref, nbr_ref, wst_ref, wet_ref, bfc_ref):
    S = jnp.dot(x_ref[...], wst_ref[...], preferred_element_type=jnp.float32) + bfc_ref[...]
    E = jnp.dot(nbr_ref[...].reshape(_TR, _HB), wet_ref[...],
                preferred_element_type=jnp.float32)
    g32 = g_ref[...]
    gf = lax.bitcast_convert_type(g32 << 16, jnp.float32).reshape(_TN, _M, _HA)
    gc = lax.bitcast_convert_type(g32 & jnp.int32(-65536), jnp.float32).reshape(_TN, _M, _HA)
    zf = gf + E[:, :_HA].reshape(_TN, _M, _HA) + S[:, :_HA][:, None, :]
    zc = gc + E[:, _HA:].reshape(_TN, _M, _HA) + S[:, _HA:][:, None, :]
    return zf.reshape(_TR, _HA), zc.reshape(_TR, _HA)


def _stats_body(g_ref, x_ref, nbr_ref, wst_ref, wet_ref, bfc_ref, acc_ref):
    i = pl.program_id(0)
    zf, zc = _z_tile(g_ref, x_ref, nbr_ref, wst_ref, wet_ref, bfc_ref)
    part = jnp.stack([jnp.concatenate([zf.sum(0), zc.sum(0)]),
                      jnp.concatenate([(zf * zf).sum(0), (zc * zc).sum(0)])])

    @pl.when(i == 0)
    def _():
        acc_ref[...] = jnp.zeros_like(acc_ref)

    acc_ref[...] += part


def _tc_stats(G, x, nbr4, WsT, WeT, bfc):
    return pl.pallas_call(
        _stats_body,
        grid=(_ROWS // _TR,),
        in_specs=[pl.BlockSpec((_TR, _HA), lambda i: (i, 0)),
                  pl.BlockSpec((_TN, _HA), lambda i: (i, 0)),
                  pl.BlockSpec((1, _TN, _M, _HB),
                               lambda i: (i // (_N // _TN), i % (_N // _TN), 0, 0)),
                  pl.BlockSpec((_HA, _C), lambda i: (0, 0)),
                  pl.BlockSpec((_HB, _C), lambda i: (0, 0)),
                  pl.BlockSpec((1, _C), lambda i: (0, 0))],
        out_specs=pl.BlockSpec((2, _C), lambda i: (0, 0)),
        out_shape=jax.ShapeDtypeStruct((2, _C), jnp.float32),
    )(G, x, nbr4, WsT, WeT, bfc)


def _apply_body(g_ref, x_ref, nbr_ref, wst_ref, wet_ref, bfc_ref, st_ref,
                gam_ref, bet_ref, s_ref, acc_ref):
    i = pl.program_id(0)
    zf, zc = _z_tile(g_ref, x_ref, nbr_ref, wst_ref, wet_ref, bfc_ref)
    mean = st_ref[0, :] * _INV_NH
    var = st_ref[1, :] * _INV_NH - mean * mean
    scale = gam_ref[0, :] * lax.rsqrt(var + _EPS)
    shift = bet_ref[0, :] - mean * scale
    f = jax.nn.sigmoid(zf * scale[:_HA] + shift[:_HA])
    c = jnp.maximum(zc * scale[_HA:] + shift[_HA:], 0.0)
    fc = f * c
    s = fc.reshape(_TN, _M, _HA).sum(axis=1)
    s_ref[...] = s
    part = jnp.stack([s.sum(0), (s * s).sum(0)])

    @pl.when(i == 0)
    def _():
        acc_ref[...] = jnp.zeros_like(acc_ref)

    acc_ref[...] += part


def _tc_apply(G, x, nbr4, WsT, WeT, bfc, stats_h, gam, bet):
    return pl.pallas_call(
        _apply_body,
        grid=(_ROWS // _TR,),
        in_specs=[pl.BlockSpec((_TR, _HA), lambda i: (i, 0)),
                  pl.BlockSpec((_TN, _HA), lambda i: (i, 0)),
                  pl.BlockSpec((1, _TN, _M, _HB),
                               lambda i: (i // (_N // _TN), i % (_N // _TN), 0, 0)),
                  pl.BlockSpec((_HA, _C), lambda i: (0, 0)),
                  pl.BlockSpec((_HB, _C), lambda i: (0, 0)),
                  pl.BlockSpec((1, _C), lambda i: (0, 0)),
                  pl.BlockSpec((2, _C), lambda i: (0, 0)),
                  pl.BlockSpec((1, _C), lambda i: (0, 0)),
                  pl.BlockSpec((1, _C), lambda i: (0, 0))],
        out_specs=[pl.BlockSpec((_TN, _HA), lambda i: (i, 0)),
                   pl.BlockSpec((2, _HA), lambda i: (0, 0))],
        out_shape=[jax.ShapeDtypeStruct((_NODES, _HA), jnp.float32),
                   jax.ShapeDtypeStruct((2, _HA), jnp.float32)],
    )(G, x, nbr4, WsT, WeT, bfc, stats_h, gam, bet)


def _final_body(x_ref, s_ref, st_ref, gam_ref, bet_ref, o_ref):
    mean = st_ref[0, :] * _INV_NO
    var = st_ref[1, :] * _INV_NO - mean * mean
    scale = gam_ref[0, :] * lax.rsqrt(var + _EPS)
    shift = bet_ref[0, :] - mean * scale
    o_ref[...] = jnp.maximum(x_ref[...] + s_ref[...] * scale + shift, 0.0)


def _tc_final(x, s, stats_o, gam, bet):
    T = 1024
    return pl.pallas_call(
        _final_body,
        grid=(_NODES // T,),
        in_specs=[pl.BlockSpec((T, _HA), lambda i: (i, 0)),
                  pl.BlockSpec((T, _HA), lambda i: (i, 0)),
                  pl.BlockSpec((2, _HA), lambda i: (0, 0)),
                  pl.BlockSpec((1, _HA), lambda i: (0, 0)),
                  pl.BlockSpec((1, _HA), lambda i: (0, 0))],
        out_specs=pl.BlockSpec((T, _HA), lambda i: (i, 0)),
        out_shape=jax.ShapeDtypeStruct((_NODES, _HA), jnp.float32),
    )(x, s, stats_o, gam, bet)


def _sc_gather(table, gidx2):
    info = plsc.get_sparse_core_info()
    nw = info.num_cores * info.num_subcores
    per_w = _ROWS // nw
    ch = 128
    nch = per_w // ch
    gpc = 2
    nsc = nch // gpc
    mesh = plsc.VectorSubcoreMesh(core_axis_name="c", subcore_axis_name="s")

    @functools.partial(
        pl.kernel, mesh=mesh,
        out_type=jax.ShapeDtypeStruct((_ROWS, _HA), jnp.int32),
        scratch_types=[
            pltpu.VMEM((nch, ch), jnp.int32),
            pltpu.VMEM((2, gpc * ch, _HA), jnp.int32),
            pltpu.SemaphoreType.DMA,
            pltpu.SemaphoreType.DMA,
        ],
    )
    def k(table_hbm, idx_hbm, out_hbm, idx_v, rows_v, sem0, sem1):
        wid = lax.axis_index("s") * info.num_cores + lax.axis_index("c")
        base = wid * per_w
        pltpu.sync_copy(idx_hbm.at[pl.ds(wid * nch, nch)], idx_v)
        sems = (sem0, sem1)

        def start(i, j):
            for g in range(gpc):
                pltpu.async_copy(table_hbm.at[idx_v.at[i * gpc + g]],
                                 rows_v.at[j, pl.ds(g * ch, ch)], sems[j])

        def drain(i, j):
            for g in range(gpc):
                pltpu.make_async_copy(table_hbm.at[idx_v.at[i * gpc + g]],
                                      rows_v.at[j, pl.ds(g * ch, ch)],
                                      sems[j]).wait()

        start(0, 0)

        def body(i, carry):
            for j in range(2):
                @pl.when(i % 2 == j)
                def _():
                    @pl.when(i + 1 < nsc)
                    def _():
                        start(i + 1, 1 - j)

                    drain(i, j)
                    pltpu.sync_copy(
                        rows_v.at[j],
                        out_hbm.at[pl.ds(base + i * gpc * ch, gpc * ch)])
            return carry

        lax.fori_loop(0, nsc, body, 0)

    return k(table, gidx2)


def kernel(atom_emb, nbr_emb, nbr_adj_list, W_fc, b_fc, gamma_h, beta_h,
           gamma_o, beta_o):
    B, N, HA = atom_emb.shape
    M = nbr_adj_list.shape[2]
    C = 2 * HA
    x = atom_emb.reshape(B * N, HA)
    WsT = W_fc[:, :HA].T
    WnT = W_fc[:, HA:2 * HA].T
    WeT = W_fc[:, 2 * HA:].T

    P32 = _tc_proj(x, WnT[:, :HA], WnT[:, HA:])
    gidx = (nbr_adj_list.astype(jnp.int32)
            + (jnp.arange(B, dtype=jnp.int32) * N)[:, None, None]).reshape(-1, 128)
    G = _sc_gather(P32, gidx)

    nbr4 = nbr_emb
    bfc = b_fc.reshape(1, C)
    stats_h = _tc_stats(G, x, nbr4, WsT, WeT, bfc)
    s, stats_o = _tc_apply(G, x, nbr4, WsT, WeT, bfc, stats_h,
                           gamma_h.reshape(1, C), beta_h.reshape(1, C))
    out = _tc_final(x, s, stats_o, gamma_o.reshape(1, HA), beta_o.reshape(1, HA))
    return out.reshape(B, N, HA)


# speedup vs baseline: 1.1134x; 1.1006x over previous
import functools

import jax
import jax.numpy as jnp
from jax import lax
from jax.experimental import pallas as pl
from jax.experimental.pallas import tpu as pltpu
from jax.experimental.pallas import tpu_sc as plsc

_B, _N, _M, _HA, _HB = 8, 1024, 16, 128, 16
_C = 2 * _HA
_ROWS = _B * _N * _M
_NODES = _B * _N
_TN = 512
_TR = _TN * _M
_INV_NH = 1.0 / _ROWS
_INV_NO = 1.0 / _NODES
_EPS = 1e-5


def _proj_body(x_ref, wf_ref, wc_ref, o_ref):
    yf = jnp.dot(x_ref[...], wf_ref[...], preferred_element_type=jnp.float32)
    yc = jnp.dot(x_ref[...], wc_ref[...], preferred_element_type=jnp.float32)
    uf = lax.bitcast_convert_type(yf.astype(jnp.bfloat16), jnp.uint16).astype(jnp.int32)
    uc = lax.bitcast_convert_type(yc.astype(jnp.bfloat16), jnp.uint16).astype(jnp.int32)
    o_ref[...] = uf | (uc << 16)


def _tc_proj(x, WnTf, WnTc):
    T = 1024
    return pl.pallas_call(
        _proj_body,
        grid=(_NODES // T,),
        in_specs=[pl.BlockSpec((T, _HA), lambda i: (i, 0)),
                  pl.BlockSpec((_HA, _HA), lambda i: (0, 0)),
                  pl.BlockSpec((_HA, _HA), lambda i: (0, 0))],
        out_specs=pl.BlockSpec((T, _HA), lambda i: (i, 0)),
        out_shape=jax.ShapeDtypeStruct((_NODES, _HA), jnp.int32),
    )(x, WnTf, WnTc)


def _z_tile(g_ref, x_ref, nbr_ref, wst_ref, wet_ref, bfc_ref):
    S = jnp.dot(x_ref[...], wst_ref[...], preferred_element_type=jnp.float32) + bfc_ref[...]
    E = jnp.dot(nbr_ref[...], wet_ref[...], preferred_element_type=jnp.float32)
    g32 = g_ref[...]
    gf = lax.bitcast_convert_type(g32 << 16, jnp.float32).reshape(_TN, _M, _HA)
    gc = lax.bitcast_convert_type(g32 & jnp.int32(-65536), jnp.float32).reshape(_TN, _M, _HA)
    zf = gf + E[:, :_HA].reshape(_TN, _M, _HA) + S[:, :_HA][:, None, :]
    zc = gc + E[:, _HA:].reshape(_TN, _M, _HA) + S[:, _HA:][:, None, :]
    return zf.reshape(_TR, _HA), zc.reshape(_TR, _HA)


def _stats_body(g_ref, x_ref, nbr_ref, wst_ref, wet_ref, bfc_ref, acc_ref):
    i = pl.program_id(0)
    zf, zc = _z_tile(g_ref, x_ref, nbr_ref, wst_ref, wet_ref, bfc_ref)
    part = jnp.stack([jnp.concatenate([zf.sum(0), zc.sum(0)]),
                      jnp.concatenate([(zf * zf).sum(0), (zc * zc).sum(0)])])

    @pl.when(i == 0)
    def _():
        acc_ref[...] = jnp.zeros_like(acc_ref)

    acc_ref[...] += part


def _tc_stats(G, x, nbr4, WsT, WeT, bfc):
    return pl.pallas_call(
        _stats_body,
        grid=(_ROWS // _TR,),
        in_specs=[pl.BlockSpec((_TR, _HA), lambda i: (i, 0)),
                  pl.BlockSpec((_TN, _HA), lambda i: (i, 0)),
                  pl.BlockSpec((_TR, _HB), lambda i: (i, 0)),
                  pl.BlockSpec((_HA, _C), lambda i: (0, 0)),
                  pl.BlockSpec((_HB, _C), lambda i: (0, 0)),
                  pl.BlockSpec((1, _C), lambda i: (0, 0))],
        out_specs=pl.BlockSpec((2, _C), lambda i: (0, 0)),
        out_shape=jax.ShapeDtypeStruct((2, _C), jnp.float32),
    )(G, x, nbr4, WsT, WeT, bfc)


def _apply_body(g_ref, x_ref, nbr_ref, wst_ref, wet_ref, bfc_ref, st_ref,
                gam_ref, bet_ref, s_ref, acc_ref):
    i = pl.program_id(0)
    zf, zc = _z_tile(g_ref, x_ref, nbr_ref, wst_ref, wet_ref, bfc_ref)
    mean = st_ref[0, :] * _INV_NH
    var = st_ref[1, :] * _INV_NH - mean * mean
    scale = gam_ref[0, :] * lax.rsqrt(var + _EPS)
    shift = bet_ref[0, :] - mean * scale
    f = jax.nn.sigmoid(zf * scale[:_HA] + shift[:_HA])
    c = jnp.maximum(zc * scale[_HA:] + shift[_HA:], 0.0)
    fc = f * c
    s = fc.reshape(_TN, _M, _HA).sum(axis=1)
    s_ref[...] = s
    part = jnp.stack([s.sum(0), (s * s).sum(0)])

    @pl.when(i == 0)
    def _():
        acc_ref[...] = jnp.zeros_like(acc_ref)

    acc_ref[...] += part


def _tc_apply(G, x, nbr4, WsT, WeT, bfc, stats_h, gam, bet):
    return pl.pallas_call(
        _apply_body,
        grid=(_ROWS // _TR,),
        in_specs=[pl.BlockSpec((_TR, _HA), lambda i: (i, 0)),
                  pl.BlockSpec((_TN, _HA), lambda i: (i, 0)),
                  pl.BlockSpec((_TR, _HB), lambda i: (i, 0)),
                  pl.BlockSpec((_HA, _C), lambda i: (0, 0)),
                  pl.BlockSpec((_HB, _C), lambda i: (0, 0)),
                  pl.BlockSpec((1, _C), lambda i: (0, 0)),
                  pl.BlockSpec((2, _C), lambda i: (0, 0)),
                  pl.BlockSpec((1, _C), lambda i: (0, 0)),
                  pl.BlockSpec((1, _C), lambda i: (0, 0))],
        out_specs=[pl.BlockSpec((_TN, _HA), lambda i: (i, 0)),
                   pl.BlockSpec((2, _HA), lambda i: (0, 0))],
        out_shape=[jax.ShapeDtypeStruct((_NODES, _HA), jnp.float32),
                   jax.ShapeDtypeStruct((2, _HA), jnp.float32)],
    )(G, x, nbr4, WsT, WeT, bfc, stats_h, gam, bet)


def _final_body(x_ref, s_ref, st_ref, gam_ref, bet_ref, o_ref):
    mean = st_ref[0, :] * _INV_NO
    var = st_ref[1, :] * _INV_NO - mean * mean
    scale = gam_ref[0, :] * lax.rsqrt(var + _EPS)
    shift = bet_ref[0, :] - mean * scale
    o_ref[...] = jnp.maximum(x_ref[...] + s_ref[...] * scale + shift, 0.0)


def _tc_final(x, s, stats_o, gam, bet):
    T = 1024
    return pl.pallas_call(
        _final_body,
        grid=(_NODES // T,),
        in_specs=[pl.BlockSpec((T, _HA), lambda i: (i, 0)),
                  pl.BlockSpec((T, _HA), lambda i: (i, 0)),
                  pl.BlockSpec((2, _HA), lambda i: (0, 0)),
                  pl.BlockSpec((1, _HA), lambda i: (0, 0)),
                  pl.BlockSpec((1, _HA), lambda i: (0, 0))],
        out_specs=pl.BlockSpec((T, _HA), lambda i: (i, 0)),
        out_shape=jax.ShapeDtypeStruct((_NODES, _HA), jnp.float32),
    )(x, s, stats_o, gam, bet)


def _sc_gather(table, gidx2):
    info = plsc.get_sparse_core_info()
    nw = info.num_cores * info.num_subcores
    per_w = _ROWS // nw
    ch = 128
    nch = per_w // ch
    gpc = 2
    nsc = nch // gpc
    mesh = plsc.VectorSubcoreMesh(core_axis_name="c", subcore_axis_name="s")

    @functools.partial(
        pl.kernel, mesh=mesh,
        out_type=jax.ShapeDtypeStruct((_ROWS, _HA), jnp.int32),
        scratch_types=[
            pltpu.VMEM((nch, ch), jnp.int32),
            pltpu.VMEM((2, gpc * ch, _HA), jnp.int32),
            pltpu.SemaphoreType.DMA,
            pltpu.SemaphoreType.DMA,
        ],
    )
    def k(table_hbm, idx_hbm, out_hbm, idx_v, rows_v, sem0, sem1):
        wid = lax.axis_index("s") * info.num_cores + lax.axis_index("c")
        base = wid * per_w
        pltpu.sync_copy(idx_hbm.at[pl.ds(wid * nch, nch)], idx_v)
        sems = (sem0, sem1)

        def start(i, j):
            for g in range(gpc):
                pltpu.async_copy(table_hbm.at[idx_v.at[i * gpc + g]],
                                 rows_v.at[j, pl.ds(g * ch, ch)], sems[j])

        def drain(i, j):
            for g in range(gpc):
                pltpu.make_async_copy(table_hbm.at[idx_v.at[i * gpc + g]],
                                      rows_v.at[j, pl.ds(g * ch, ch)],
                                      sems[j]).wait()

        start(0, 0)

        def body(i, carry):
            for j in range(2):
                @pl.when(i % 2 == j)
                def _():
                    @pl.when(i + 1 < nsc)
                    def _():
                        start(i + 1, 1 - j)

                    drain(i, j)
                    pltpu.sync_copy(
                        rows_v.at[j],
                        out_hbm.at[pl.ds(base + i * gpc * ch, gpc * ch)])
            return carry

        lax.fori_loop(0, nsc, body, 0)

    return k(table, gidx2)


def kernel(atom_emb, nbr_emb, nbr_adj_list, W_fc, b_fc, gamma_h, beta_h,
           gamma_o, beta_o):
    B, N, HA = atom_emb.shape
    M = nbr_adj_list.shape[2]
    C = 2 * HA
    x = atom_emb.reshape(B * N, HA)
    WsT = W_fc[:, :HA].T
    WnT = W_fc[:, HA:2 * HA].T
    WeT = W_fc[:, 2 * HA:].T

    P32 = _tc_proj(x, WnT[:, :HA], WnT[:, HA:])
    gidx = (nbr_adj_list.astype(jnp.int32)
            + (jnp.arange(B, dtype=jnp.int32) * N)[:, None, None]).reshape(-1, 128)
    G = _sc_gather(P32, gidx)

    nbr4 = nbr_emb.reshape(B * N * M, -1)
    bfc = b_fc.reshape(1, C)
    stats_h = _tc_stats(G, x, nbr4, WsT, WeT, bfc)
    s, stats_o = _tc_apply(G, x, nbr4, WsT, WeT, bfc, stats_h,
                           gamma_h.reshape(1, C), beta_h.reshape(1, C))
    out = _tc_final(x, s, stats_o, gamma_o.reshape(1, HA), beta_o.reshape(1, HA))
    return out.reshape(B, N, HA)
